# Initial kernel scaffold; baseline (speedup 1.0000x reference)
import functools

import jax
import jax.numpy as jnp
from jax import lax
from jax.experimental import pallas as pl
from jax.experimental.pallas import tpu as pltpu
from jax.experimental.pallas import tpu_sc as plsc


def _leaky(v):
    return jnp.where(v >= 0, v, 0.15 * v)


def _he_body(ea, w1, b1, w2, b2, he1, he2):
    e = ea[...]
    h1 = jnp.dot(e, w1[...], preferred_element_type=jnp.float32,
                 precision=lax.Precision.HIGHEST) + b1[...]
    he1[...] = _leaky(h1)
    h2 = jnp.dot(e, w2[...], preferred_element_type=jnp.float32,
                 precision=lax.Precision.HIGHEST) + b2[...]
    he2[...] = _leaky(h2)


def _edge_mlp(ea, w1t, b1, w2t, b2, be=3200):
    E, DE = ea.shape
    H = w1t.shape[1]
    grid = E // be
    return pl.pallas_call(
        _he_body,
        grid=(grid,),
        in_specs=[
            pl.BlockSpec((be, DE), lambda i: (i, 0)),
            pl.BlockSpec((DE, H), lambda i: (0, 0)),
            pl.BlockSpec((1, H), lambda i: (0, 0)),
            pl.BlockSpec((DE, H), lambda i: (0, 0)),
            pl.BlockSpec((1, H), lambda i: (0, 0)),
        ],
        out_specs=[
            pl.BlockSpec((be, H), lambda i: (i, 0)),
            pl.BlockSpec((be, H), lambda i: (i, 0)),
        ],
        out_shape=[
            jax.ShapeDtypeStruct((E, H), jnp.float32),
            jax.ShapeDtypeStruct((E, H), jnp.float32),
        ],
    )(ea, w1t, b1, w2t, b2)


_CH = 80
_DW = 16


def _segsum_sc(he1, he2, dst, npad):
    E, H = he1.shape
    ns = 16
    ept = E // ns
    nch = ept // _CH
    rpt = npad // ns
    zr = 160

    mesh = plsc.VectorSubcoreMesh(core_axis_name="c", subcore_axis_name="s")

    @functools.partial(
        pl.kernel,
        out_type=[
            jax.ShapeDtypeStruct((npad, H), jnp.float32),
            jax.ShapeDtypeStruct((npad, H), jnp.float32),
            jax.ShapeDtypeStruct((npad, _DW), jnp.float32),
        ],
        mesh=mesh,
        scratch_types=[
            pltpu.VMEM((_CH,), jnp.int32),
            pltpu.VMEM((_CH, H), jnp.float32),
            pltpu.VMEM((_CH, _DW), jnp.float32),
            pltpu.VMEM((zr, H), jnp.float32),
            pltpu.VMEM((npad // 16, _DW), jnp.float32),
            pltpu.VMEM_SHARED((npad, H), jnp.float32),
            pltpu.VMEM_SHARED((npad, _DW), jnp.float32),
        ],
    )
    def seg(he1_h, he2_h, dst_h, s1_o, s2_o, deg_o,
            idx_v, rows_v, ones_v, zbuf, dzbuf, acc, dacc):
        cid = lax.axis_index("c")
        sid = lax.axis_index("s")
        zero16 = jnp.zeros((16,), jnp.float32)
        one16 = jnp.ones((16,), jnp.float32)

        def zrow(i, _):
            for k in range(H // 16):
                zbuf[i, pl.ds(k * 16, 16)] = zero16
            return 0
        lax.fori_loop(0, zr, zrow, 0)

        def drow(i, _):
            dzbuf[i, :] = zero16
            return 0
        lax.fori_loop(0, rpt, drow, 0)

        def orow(i, _):
            ones_v[i, :] = one16
            return 0
        lax.fori_loop(0, _CH, orow, 0)

        base_r = sid * rpt
        for z in range(rpt // zr):
            pltpu.sync_copy(zbuf, acc.at[pl.ds(base_r + z * zr, zr)])
        pltpu.sync_copy(dzbuf, dacc.at[pl.ds(base_r, rpt)])
        plsc.subcore_barrier()

        ebase = sid * ept

        def body(ci, _):
            off = ebase + ci * _CH
            pltpu.sync_copy(dst_h.at[pl.ds(off, _CH)], idx_v)

            @pl.when(cid == 0)
            def _():
                pltpu.sync_copy(he1_h.at[pl.ds(off, _CH)], rows_v)

            @pl.when(cid == 1)
            def _():
                pltpu.sync_copy(he2_h.at[pl.ds(off, _CH)], rows_v)

            pltpu.sync_copy(rows_v, acc.at[idx_v], add=True)

            @pl.when(cid == 0)
            def _():
                pltpu.sync_copy(ones_v, dacc.at[idx_v], add=True)
            return 0

        lax.fori_loop(0, nch, body, 0)
        plsc.subcore_barrier()

        @pl.when(cid == 0)
        def _():
            pltpu.sync_copy(acc.at[pl.ds(base_r, rpt)],
                            s1_o.at[pl.ds(base_r, rpt)])
            pltpu.sync_copy(dacc.at[pl.ds(base_r, rpt)],
                            deg_o.at[pl.ds(base_r, rpt)])

        @pl.when(cid == 1)
        def _():
            pltpu.sync_copy(acc.at[pl.ds(base_r, rpt)],
                            s2_o.at[pl.ds(base_r, rpt)])

    return seg(he1, he2, dst)


def _node_body(x, s1, s2, deg,
               ew2t1, eb2_1, nwxt1, nwet1, nb1,
               ew2t2, eb2_2, nwxt2, nwet2, nb2,
               muwt, mub, lvwt, lvb, mu, lv):
    def dot(a, b):
        return jnp.dot(a, b[...], preferred_element_type=jnp.float32,
                       precision=lax.Precision.HIGHEST)

    d = deg[...]
    u1 = dot(s1[...], ew2t1) + d * eb2_1[...]
    agg1 = d * dot(x[...], nwxt1) + dot(u1, nwet1) + d * nb1[...]
    h1 = _leaky(agg1)
    u2 = dot(s2[...], ew2t2) + d * eb2_2[...]
    agg2 = d * dot(h1, nwxt2) + dot(u2, nwet2) + d * nb2[...]
    h2 = _leaky(agg2)
    mu[...] = dot(h2, muwt) + mub[...]
    lv[...] = dot(h2, lvwt) + lvb[...]


def _node_stage(x, s1, s2, deg, wts, bn=1000):
    N, D = x.shape
    H = s1.shape[1]
    L = wts["muwt"].shape[1]
    grid = N // bn

    def full(shape):
        return pl.BlockSpec(shape, lambda i: (0, 0))

    return pl.pallas_call(
        _node_body,
        grid=(grid,),
        in_specs=[
            pl.BlockSpec((bn, D), lambda i: (i, 0)),
            pl.BlockSpec((bn, H), lambda i: (i, 0)),
            pl.BlockSpec((bn, H), lambda i: (i, 0)),
            pl.BlockSpec((bn, 1), lambda i: (i, 0)),
            full((H, H)), full((1, H)), full((D, H)), full((H, H)),
            full((1, H)),
            full((H, H)), full((1, H)), full((H, H)), full((H, H)),
            full((1, H)),
            full((H, L)), full((1, L)), full((H, L)), full((1, L)),
        ],
        out_specs=[
            pl.BlockSpec((bn, L), lambda i: (i, 0)),
            pl.BlockSpec((bn, L), lambda i: (i, 0)),
        ],
        out_shape=[
            jax.ShapeDtypeStruct((N, L), jnp.float32),
            jax.ShapeDtypeStruct((N, L), jnp.float32),
        ],
    )(x, s1, s2, deg,
      wts["ew2t1"], wts["eb2_1"], wts["nwxt1"], wts["nwet1"], wts["nb1"],
      wts["ew2t2"], wts["eb2_2"], wts["nwxt2"], wts["nwet2"], wts["nb2"],
      wts["muwt"], wts["mub"], wts["lvwt"], wts["lvb"])


def kernel(x, edge_index, edge_attr,
           c1_ew1, c1_eb1, c1_ew2, c1_eb2, c1_nw, c1_nb,
           c2_ew1, c2_eb1, c2_ew2, c2_eb2, c2_nw, c2_nb,
           mu_w, mu_b, lv_w, lv_b):
    N, D = x.shape
    H = c1_ew1.shape[0]
    dst = edge_index[1].astype(jnp.int32)

    he1, he2 = _edge_mlp(edge_attr,
                         c1_ew1.T, c1_eb1[None, :],
                         c2_ew1.T, c2_eb1[None, :])

    npad = ((N + 2047) // 2048) * 2048
    s1f, s2f, degf = _segsum_sc(he1, he2, dst, npad)
    s1 = s1f[:N]
    s2 = s2f[:N]
    deg = degf[:N, 0:1]

    wts = dict(
        ew2t1=c1_ew2.T, eb2_1=c1_eb2[None, :],
        nwxt1=c1_nw[:, :D].T, nwet1=c1_nw[:, D:].T, nb1=c1_nb[None, :],
        ew2t2=c2_ew2.T, eb2_2=c2_eb2[None, :],
        nwxt2=c2_nw[:, :H].T, nwet2=c2_nw[:, H:].T, nb2=c2_nb[None, :],
        muwt=mu_w.T, mub=mu_b[None, :],
        lvwt=lv_w.T, lvb=lv_b[None, :],
    )
    mu, lv = _node_stage(x, s1, s2, deg, wts)
    return mu, lv


# speedup vs baseline: 2.9619x; 2.9619x over previous
import functools

import jax
import jax.numpy as jnp
from jax import lax
from jax.experimental import pallas as pl
from jax.experimental.pallas import tpu as pltpu
from jax.experimental.pallas import tpu_sc as plsc


def _leaky(v):
    return jnp.where(v >= 0, v, 0.15 * v)


def _he_body(ea, w1, b1, w2, b2, he1, he2):
    e = ea[...]
    h1 = jnp.dot(e, w1[...], preferred_element_type=jnp.float32,
                 precision=lax.Precision.HIGHEST) + b1[...]
    he1[...] = _leaky(h1)
    h2 = jnp.dot(e, w2[...], preferred_element_type=jnp.float32,
                 precision=lax.Precision.HIGHEST) + b2[...]
    he2[...] = _leaky(h2)


def _edge_mlp(ea, w1t, b1, w2t, b2, be=3200):
    E, DE = ea.shape
    H = w1t.shape[1]
    grid = E // be
    return pl.pallas_call(
        _he_body,
        grid=(grid,),
        in_specs=[
            pl.BlockSpec((be, DE), lambda i: (i, 0)),
            pl.BlockSpec((DE, H), lambda i: (0, 0)),
            pl.BlockSpec((1, H), lambda i: (0, 0)),
            pl.BlockSpec((DE, H), lambda i: (0, 0)),
            pl.BlockSpec((1, H), lambda i: (0, 0)),
        ],
        out_specs=[
            pl.BlockSpec((be, H), lambda i: (i, 0)),
            pl.BlockSpec((be, H), lambda i: (i, 0)),
        ],
        out_shape=[
            jax.ShapeDtypeStruct((E, H), jnp.float32),
            jax.ShapeDtypeStruct((E, H), jnp.float32),
        ],
    )(ea, w1t, b1, w2t, b2)


_CH = 80
_DW = 16


def _segsum_one(he, dst, npad, lo, with_deg):
    E, H = he.shape
    ns = 16
    half = npad // 2
    arows = half + 128
    zrt = arows // ns
    wrt = half // ns
    drt = npad // ns
    ept = E // ns
    grp = 128
    ngrp = ept // grp
    tail = ept - ngrp * grp
    zr = 164
    assert zrt == 2 * zr and tail % 8 == 0

    mesh = plsc.VectorSubcoreMesh(core_axis_name="c", subcore_axis_name="s")
    out_type = [jax.ShapeDtypeStruct((npad, H), jnp.float32)]
    if with_deg:
        out_type.append(jax.ShapeDtypeStruct((npad,), jnp.float32))

    scratch = [
        pltpu.VMEM((grp,), jnp.int32),
        pltpu.VMEM((grp,), jnp.int32),
        pltpu.VMEM((grp, H), jnp.float32),
        pltpu.VMEM((tail,), jnp.int32),
        pltpu.VMEM((tail,), jnp.int32),
        pltpu.VMEM((tail, H), jnp.float32),
        pltpu.VMEM((zr, H), jnp.float32),
        pltpu.VMEM_SHARED((arows, H), jnp.float32),
    ]
    if with_deg:
        scratch += [
            pltpu.VMEM((grp,), jnp.float32),
            pltpu.VMEM((drt,), jnp.float32),
            pltpu.VMEM_SHARED((npad,), jnp.float32),
        ]

    @functools.partial(pl.kernel, out_type=out_type, mesh=mesh,
                       scratch_types=scratch)
    def seg(he_h, dst_h, *refs):
        if with_deg:
            (s_o, deg_o, idx_v, idx2_v, rows_v, idxt_v, idxt2_v, rowst_v,
             zbuf, acc, ones1, dz, dacc) = refs
        else:
            (s_o, idx_v, idx2_v, rows_v, idxt_v, idxt2_v, rowst_v,
             zbuf, acc) = refs
        cid = lax.axis_index("c")
        sid = lax.axis_index("s")
        zero16 = jnp.zeros((16,), jnp.float32)
        one16 = jnp.ones((16,), jnp.float32)
        lo_c = lo + cid * half

        def zrow(i, _):
            for k in range(H // 16):
                zbuf[i, pl.ds(k * 16, 16)] = zero16
            return 0
        lax.fori_loop(0, zr, zrow, 0)

        zbase = sid * zrt
        pltpu.sync_copy(zbuf, acc.at[pl.ds(zbase, zr)])
        pltpu.sync_copy(zbuf, acc.at[pl.ds(zbase + zr, zr)])

        if with_deg:
            def frow(i, _):
                dz[pl.ds(i * 16, 16)] = zero16
                return 0
            lax.fori_loop(0, drt // 16, frow, 0)

            def orow(i, _):
                ones1[pl.ds(i * 16, 16)] = one16
                return 0
            lax.fori_loop(0, grp // 16, orow, 0)
            pltpu.sync_copy(dz, dacc.at[pl.ds(sid * drt, drt)])

        plsc.subcore_barrier()

        def route(src_ref, dst_ref, n):
            for k in range(n // 16):
                iv = src_ref[pl.ds(k * 16, 16)]
                t = iv - lo_c
                m = (t >= 0) & (t < half)
                dst_ref[pl.ds(k * 16, 16)] = jnp.where(
                    m, t, half + (iv & 63))

        ebase = sid * ept

        def body(ci, _):
            off = ebase + ci * grp
            pltpu.sync_copy(dst_h.at[pl.ds(off, grp)], idx_v)
            pltpu.sync_copy(he_h.at[pl.ds(off, grp)], rows_v)
            route(idx_v, idx2_v, grp)
            pltpu.sync_copy(rows_v, acc.at[idx2_v], add=True)
            if with_deg:
                @pl.when(cid == 0)
                def _():
                    pltpu.sync_copy(ones1, dacc.at[idx_v], add=True)
            return 0

        lax.fori_loop(0, ngrp, body, 0)

        if tail:
            off = ebase + ngrp * grp
            pltpu.sync_copy(dst_h.at[pl.ds(off, tail)], idxt_v)
            pltpu.sync_copy(he_h.at[pl.ds(off, tail)], rowst_v)
            route(idxt_v, idxt2_v, tail)
            pltpu.sync_copy(rowst_v, acc.at[idxt2_v], add=True)
            if with_deg:
                @pl.when(cid == 0)
                def _():
                    pltpu.sync_copy(ones1.at[pl.ds(0, tail)],
                                    dacc.at[idxt_v], add=True)

        plsc.subcore_barrier()

        pltpu.sync_copy(acc.at[pl.ds(sid * wrt, wrt)],
                        s_o.at[pl.ds(lo_c + sid * wrt, wrt)])
        if with_deg:
            @pl.when(cid == 0)
            def _():
                pltpu.sync_copy(dacc.at[pl.ds(sid * drt, drt)],
                                deg_o.at[pl.ds(sid * drt, drt)])

    return seg(he, dst)


def _segsum_sc(he1, he2, dst, npad):
    s1, deg = _segsum_one(he1, dst, npad, 0, True)
    (s2,) = _segsum_one(he2, dst, npad, 0, False)
    return s1, s2, deg


def _node_body(x, s1, s2, deg,
               ew2t1, eb2_1, nwxt1, nwet1, nb1,
               ew2t2, eb2_2, nwxt2, nwet2, nb2,
               muwt, mub, lvwt, lvb, mu, lv):
    def dot(a, b):
        return jnp.dot(a, b[...], preferred_element_type=jnp.float32,
                       precision=lax.Precision.HIGHEST)

    d = deg[...]
    u1 = dot(s1[...], ew2t1) + d * eb2_1[...]
    agg1 = d * dot(x[...], nwxt1) + dot(u1, nwet1) + d * nb1[...]
    h1 = _leaky(agg1)
    u2 = dot(s2[...], ew2t2) + d * eb2_2[...]
    agg2 = d * dot(h1, nwxt2) + dot(u2, nwet2) + d * nb2[...]
    h2 = _leaky(agg2)
    mu[...] = dot(h2, muwt) + mub[...]
    lv[...] = dot(h2, lvwt) + lvb[...]


def _node_stage(x, s1, s2, deg, wts, bn=1000):
    N, D = x.shape
    H = s1.shape[1]
    L = wts["muwt"].shape[1]
    grid = N // bn

    def full(shape):
        return pl.BlockSpec(shape, lambda i: (0, 0))

    return pl.pallas_call(
        _node_body,
        grid=(grid,),
        in_specs=[
            pl.BlockSpec((bn, D), lambda i: (i, 0)),
            pl.BlockSpec((bn, H), lambda i: (i, 0)),
            pl.BlockSpec((bn, H), lambda i: (i, 0)),
            pl.BlockSpec((bn, 1), lambda i: (i, 0)),
            full((H, H)), full((1, H)), full((D, H)), full((H, H)),
            full((1, H)),
            full((H, H)), full((1, H)), full((H, H)), full((H, H)),
            full((1, H)),
            full((H, L)), full((1, L)), full((H, L)), full((1, L)),
        ],
        out_specs=[
            pl.BlockSpec((bn, L), lambda i: (i, 0)),
            pl.BlockSpec((bn, L), lambda i: (i, 0)),
        ],
        out_shape=[
            jax.ShapeDtypeStruct((N, L), jnp.float32),
            jax.ShapeDtypeStruct((N, L), jnp.float32),
        ],
    )(x, s1, s2, deg,
      wts["ew2t1"], wts["eb2_1"], wts["nwxt1"], wts["nwet1"], wts["nb1"],
      wts["ew2t2"], wts["eb2_2"], wts["nwxt2"], wts["nwet2"], wts["nb2"],
      wts["muwt"], wts["mub"], wts["lvwt"], wts["lvb"])


def kernel(x, edge_index, edge_attr,
           c1_ew1, c1_eb1, c1_ew2, c1_eb2, c1_nw, c1_nb,
           c2_ew1, c2_eb1, c2_ew2, c2_eb2, c2_nw, c2_nb,
           mu_w, mu_b, lv_w, lv_b):
    N, D = x.shape
    H = c1_ew1.shape[0]
    dst = edge_index[1].astype(jnp.int32)

    he1, he2 = _edge_mlp(edge_attr,
                         c1_ew1.T, c1_eb1[None, :],
                         c2_ew1.T, c2_eb1[None, :])

    npad = ((N + 2047) // 2048) * 2048
    s1f, s2f, degf = _segsum_sc(he1, he2, dst, npad)
    s1 = s1f[:N]
    s2 = s2f[:N]
    deg = degf[:N][:, None]

    wts = dict(
        ew2t1=c1_ew2.T, eb2_1=c1_eb2[None, :],
        nwxt1=c1_nw[:, :D].T, nwet1=c1_nw[:, D:].T, nb1=c1_nb[None, :],
        ew2t2=c2_ew2.T, eb2_2=c2_eb2[None, :],
        nwxt2=c2_nw[:, :H].T, nwet2=c2_nw[:, H:].T, nb2=c2_nb[None, :],
        muwt=mu_w.T, mub=mu_b[None, :],
        lvwt=lv_w.T, lvb=lv_b[None, :],
    )
    mu, lv = _node_stage(x, s1, s2, deg, wts)
    return mu, lv


# speedup vs baseline: 3.2791x; 1.1071x over previous
import functools

import jax
import jax.numpy as jnp
from jax import lax
from jax.experimental import pallas as pl
from jax.experimental.pallas import tpu as pltpu
from jax.experimental.pallas import tpu_sc as plsc


def _leaky(v):
    return jnp.where(v >= 0, v, 0.15 * v)


def _he_body(ea, w1, b1, w2, b2, he1, he2):
    e = ea[...]
    h1 = jnp.dot(e, w1[...], preferred_element_type=jnp.float32,
                 precision=lax.Precision.HIGHEST) + b1[...]
    he1[...] = _leaky(h1)
    h2 = jnp.dot(e, w2[...], preferred_element_type=jnp.float32,
                 precision=lax.Precision.HIGHEST) + b2[...]
    he2[...] = _leaky(h2)


def _edge_mlp(ea, w1t, b1, w2t, b2, be=3200):
    E, DE = ea.shape
    H = w1t.shape[1]
    grid = E // be
    return pl.pallas_call(
        _he_body,
        grid=(grid,),
        in_specs=[
            pl.BlockSpec((be, DE), lambda i: (i, 0)),
            pl.BlockSpec((DE, H), lambda i: (0, 0)),
            pl.BlockSpec((1, H), lambda i: (0, 0)),
            pl.BlockSpec((DE, H), lambda i: (0, 0)),
            pl.BlockSpec((1, H), lambda i: (0, 0)),
        ],
        out_specs=[
            pl.BlockSpec((be, H), lambda i: (i, 0)),
            pl.BlockSpec((be, H), lambda i: (i, 0)),
        ],
        out_shape=[
            jax.ShapeDtypeStruct((E, H), jnp.float32),
            jax.ShapeDtypeStruct((E, H), jnp.float32),
        ],
    )(ea, w1t, b1, w2t, b2)


_CH = 80
_DW = 16


def _segsum_one(he, dst, npad, with_deg):
    E, H = he.shape
    ns = 16
    half = npad // 2
    arows = half + 32
    zrt = arows // ns
    wrt = half // ns
    grp = 128
    tg = E // grp
    ng = tg // ns
    xg = tg - ng * ns
    zr = zrt // 2

    mesh = plsc.VectorSubcoreMesh(core_axis_name="c", subcore_axis_name="s")
    out_type = [jax.ShapeDtypeStruct((npad, H), jnp.float32)]
    scratch = [
        pltpu.VMEM(((ng + 1) * grp,), jnp.int32),
        pltpu.VMEM((ng + 1, grp), jnp.int32),
        pltpu.VMEM((grp, H), jnp.float32),
        pltpu.VMEM((grp, H), jnp.float32),
        pltpu.VMEM((zr, H), jnp.float32),
        pltpu.VMEM_SHARED((arows, H), jnp.float32),
    ]
    if with_deg:
        out_type.append(jax.ShapeDtypeStruct((npad,), jnp.float32))
        scratch += [
            pltpu.VMEM((grp,), jnp.float32),
            pltpu.VMEM((640,), jnp.float32),
            pltpu.VMEM_SHARED((half + 128,), jnp.float32),
        ]

    @functools.partial(pl.kernel, out_type=out_type, mesh=mesh,
                       scratch_types=scratch)
    def seg(he_h, dst_h, *refs):
        if with_deg:
            (s_o, deg_o, idx_all, idx2_all, rows_a, rows_b, zbuf, acc,
             ones1, dz, dacc) = refs
        else:
            (s_o, idx_all, idx2_all, rows_a, rows_b, zbuf, acc) = refs
        cid = lax.axis_index("c")
        sid = lax.axis_index("s")
        zero16 = jnp.zeros((16,), jnp.float32)
        one16 = jnp.ones((16,), jnp.float32)
        lo_c = cid * half
        gb = sid * ng
        xgrp = ns * ng + sid
        has_x = sid < xg

        def zrow(i, _):
            for k in range(H // 16):
                zbuf[i, pl.ds(k * 16, 16)] = zero16
            return 0
        lax.fori_loop(0, zr, zrow, 0)

        if with_deg:
            def frow(i, _):
                dz[pl.ds(i * 16, 16)] = zero16
                return 0
            lax.fori_loop(0, 640 // 16, frow, 0)

            def orow(i, _):
                ones1[pl.ds(i * 16, 16)] = one16
                return 0
            lax.fori_loop(0, grp // 16, orow, 0)

        ich = (ng // 13) * grp

        def iload(i, _):
            pltpu.sync_copy(dst_h.at[pl.ds(gb * grp + i * ich, ich)],
                            idx_all.at[pl.ds(i * ich, ich)])
            return 0
        lax.fori_loop(0, 13, iload, 0)

        @pl.when(has_x)
        def _():
            pltpu.sync_copy(dst_h.at[pl.ds(xgrp * grp, grp)],
                            idx_all.at[pl.ds(ng * grp, grp)])

        def route(i, _):
            for k in range(grp // 16):
                iv = idx_all[pl.ds(i * grp + k * 16, 16)]
                t = iv - lo_c
                m = (t >= 0) & (t < half)
                idx2_all[i, pl.ds(k * 16, 16)] = jnp.where(
                    m, t, half + (iv & 31))
            return 0
        lax.fori_loop(0, ng + 1, route, 0)

        zbase = sid * zrt
        pltpu.sync_copy(zbuf, acc.at[pl.ds(zbase, zr)])
        pltpu.sync_copy(zbuf, acc.at[pl.ds(zbase + zr, zr)])

        if with_deg:
            @pl.when(sid < half // 640)
            def _():
                pltpu.sync_copy(dz, dacc.at[pl.ds(sid * 640, 640)])

            @pl.when(sid == 8)
            def _():
                pltpu.sync_copy(dz.at[pl.ds(0, 128)],
                                dacc.at[pl.ds(half, 128)])

        plsc.subcore_barrier()

        def dscat(g):
            if with_deg:
                pltpu.sync_copy(ones1, dacc.at[idx2_all.at[g]], add=True)

        def body(g, _):
            pltpu.sync_copy(he_h.at[pl.ds((gb + g) * grp, grp)], rows_a)
            pltpu.sync_copy(rows_a, acc.at[idx2_all.at[g]], add=True)
            dscat(g)
            return 0

        lax.fori_loop(0, ng, body, 0)

        @pl.when(has_x)
        def _():
            pltpu.sync_copy(he_h.at[pl.ds(xgrp * grp, grp)], rows_a)
            pltpu.sync_copy(rows_a, acc.at[idx2_all.at[ng]], add=True)
            if with_deg:
                pltpu.sync_copy(ones1, dacc.at[idx2_all.at[ng]], add=True)

        plsc.subcore_barrier()

        pltpu.sync_copy(acc.at[pl.ds(sid * wrt, wrt)],
                        s_o.at[pl.ds(lo_c + sid * wrt, wrt)])
        if with_deg:
            @pl.when(sid < half // 640)
            def _():
                pltpu.sync_copy(dacc.at[pl.ds(sid * 640, 640)],
                                deg_o.at[pl.ds(lo_c + sid * 640, 640)])

    return seg(he, dst)


def _segsum_sc(he1, he2, dst, npad):
    s1, deg = _segsum_one(he1, dst, npad, True)
    (s2,) = _segsum_one(he2, dst, npad, False)
    return s1, s2, deg


def _node_body(x, s1, s2, deg,
               ew2t1, eb2_1, nwxt1, nwet1, nb1,
               ew2t2, eb2_2, nwxt2, nwet2, nb2,
               muwt, mub, lvwt, lvb, mu, lv):
    def dot(a, b):
        return jnp.dot(a, b[...], preferred_element_type=jnp.float32,
                       precision=lax.Precision.HIGHEST)

    d = deg[...]
    u1 = dot(s1[...], ew2t1) + d * eb2_1[...]
    agg1 = d * dot(x[...], nwxt1) + dot(u1, nwet1) + d * nb1[...]
    h1 = _leaky(agg1)
    u2 = dot(s2[...], ew2t2) + d * eb2_2[...]
    agg2 = d * dot(h1, nwxt2) + dot(u2, nwet2) + d * nb2[...]
    h2 = _leaky(agg2)
    mu[...] = dot(h2, muwt) + mub[...]
    lv[...] = dot(h2, lvwt) + lvb[...]


def _node_stage(x, s1, s2, deg, wts, bn=1000):
    N, D = x.shape
    H = s1.shape[1]
    L = wts["muwt"].shape[1]
    grid = N // bn

    def full(shape):
        return pl.BlockSpec(shape, lambda i: (0, 0))

    return pl.pallas_call(
        _node_body,
        grid=(grid,),
        in_specs=[
            pl.BlockSpec((bn, D), lambda i: (i, 0)),
            pl.BlockSpec((bn, H), lambda i: (i, 0)),
            pl.BlockSpec((bn, H), lambda i: (i, 0)),
            pl.BlockSpec((bn, 1), lambda i: (i, 0)),
            full((H, H)), full((1, H)), full((D, H)), full((H, H)),
            full((1, H)),
            full((H, H)), full((1, H)), full((H, H)), full((H, H)),
            full((1, H)),
            full((H, L)), full((1, L)), full((H, L)), full((1, L)),
        ],
        out_specs=[
            pl.BlockSpec((bn, L), lambda i: (i, 0)),
            pl.BlockSpec((bn, L), lambda i: (i, 0)),
        ],
        out_shape=[
            jax.ShapeDtypeStruct((N, L), jnp.float32),
            jax.ShapeDtypeStruct((N, L), jnp.float32),
        ],
    )(x, s1, s2, deg,
      wts["ew2t1"], wts["eb2_1"], wts["nwxt1"], wts["nwet1"], wts["nb1"],
      wts["ew2t2"], wts["eb2_2"], wts["nwxt2"], wts["nwet2"], wts["nb2"],
      wts["muwt"], wts["mub"], wts["lvwt"], wts["lvb"])


def kernel(x, edge_index, edge_attr,
           c1_ew1, c1_eb1, c1_ew2, c1_eb2, c1_nw, c1_nb,
           c2_ew1, c2_eb1, c2_ew2, c2_eb2, c2_nw, c2_nb,
           mu_w, mu_b, lv_w, lv_b):
    N, D = x.shape
    H = c1_ew1.shape[0]
    dst = edge_index[1].astype(jnp.int32)

    he1, he2 = _edge_mlp(edge_attr,
                         c1_ew1.T, c1_eb1[None, :],
                         c2_ew1.T, c2_eb1[None, :])

    npad = ((N + 2047) // 2048) * 2048
    s1f, s2f, degf = _segsum_sc(he1, he2, dst, npad)
    s1 = s1f[:N]
    s2 = s2f[:N]
    deg = degf[:N][:, None]

    wts = dict(
        ew2t1=c1_ew2.T, eb2_1=c1_eb2[None, :],
        nwxt1=c1_nw[:, :D].T, nwet1=c1_nw[:, D:].T, nb1=c1_nb[None, :],
        ew2t2=c2_ew2.T, eb2_2=c2_eb2[None, :],
        nwxt2=c2_nw[:, :H].T, nwet2=c2_nw[:, H:].T, nb2=c2_nb[None, :],
        muwt=mu_w.T, mub=mu_b[None, :],
        lvwt=lv_w.T, lvb=lv_b[None, :],
    )
    mu, lv = _node_stage(x, s1, s2, deg, wts)
    return mu, lv


# speedup vs baseline: 3.3177x; 1.0118x over previous
import functools

import jax
import jax.numpy as jnp
from jax import lax
from jax.experimental import pallas as pl
from jax.experimental.pallas import tpu as pltpu
from jax.experimental.pallas import tpu_sc as plsc


def _leaky(v):
    return jnp.where(v >= 0, v, 0.15 * v)


def _he_body(ea, w1, b1, w2, b2, he1, he2):
    e = ea[...]
    h1 = jnp.dot(e, w1[...], preferred_element_type=jnp.float32,
                 precision=lax.Precision.HIGHEST) + b1[...]
    he1[...] = _leaky(h1)
    h2 = jnp.dot(e, w2[...], preferred_element_type=jnp.float32,
                 precision=lax.Precision.HIGHEST) + b2[...]
    he2[...] = _leaky(h2)


def _edge_mlp(ea, w1t, b1, w2t, b2, be=3200):
    E, DE = ea.shape
    H = w1t.shape[1]
    grid = E // be
    return pl.pallas_call(
        _he_body,
        grid=(grid,),
        in_specs=[
            pl.BlockSpec((be, DE), lambda i: (i, 0)),
            pl.BlockSpec((DE, H), lambda i: (0, 0)),
            pl.BlockSpec((1, H), lambda i: (0, 0)),
            pl.BlockSpec((DE, H), lambda i: (0, 0)),
            pl.BlockSpec((1, H), lambda i: (0, 0)),
        ],
        out_specs=[
            pl.BlockSpec((be, H), lambda i: (i, 0)),
            pl.BlockSpec((be, H), lambda i: (i, 0)),
        ],
        out_shape=[
            jax.ShapeDtypeStruct((E, H), jnp.float32),
            jax.ShapeDtypeStruct((E, H), jnp.float32),
        ],
    )(ea, w1t, b1, w2t, b2)


_CH = 80
_DW = 16


def _segsum_sc(he1, he2, dst, npad):
    E, H = he1.shape
    ns = 16
    half = npad // 2
    arows = half + 32
    zrt = arows // ns
    wrt = half // ns
    grp = 128
    tg = E // grp
    ng = tg // ns
    xg = tg - ng * ns
    zr = zrt // 2

    mesh = plsc.VectorSubcoreMesh(core_axis_name="c", subcore_axis_name="s")
    out_type = [
        jax.ShapeDtypeStruct((npad, H), jnp.float32),
        jax.ShapeDtypeStruct((npad, H), jnp.float32),
        jax.ShapeDtypeStruct((npad,), jnp.float32),
    ]
    scratch = [
        pltpu.VMEM(((ng + 1) * grp,), jnp.int32),
        pltpu.VMEM((ng + 1, grp), jnp.int32),
        pltpu.VMEM((grp, H), jnp.float32),
        pltpu.VMEM((zr, H), jnp.float32),
        pltpu.VMEM((grp,), jnp.float32),
        pltpu.VMEM((640,), jnp.float32),
        pltpu.VMEM_SHARED((arows, H), jnp.float32),
        pltpu.VMEM_SHARED((half + 128,), jnp.float32),
    ]

    @functools.partial(pl.kernel, out_type=out_type, mesh=mesh,
                       scratch_types=scratch)
    def seg(he1_h, he2_h, dst_h, s1_o, s2_o, deg_o,
            idx_all, idx2_all, rows_a, zbuf, ones1, dz, acc, dacc):
        cid = lax.axis_index("c")
        sid = lax.axis_index("s")
        zero16 = jnp.zeros((16,), jnp.float32)
        one16 = jnp.ones((16,), jnp.float32)
        lo_c = cid * half
        gb = sid * ng
        xgrp = ns * ng + sid
        has_x = sid < xg

        def zrow(i, _):
            for k in range(H // 16):
                zbuf[i, pl.ds(k * 16, 16)] = zero16
            return 0
        lax.fori_loop(0, zr, zrow, 0)

        def frow(i, _):
            dz[pl.ds(i * 16, 16)] = zero16
            return 0
        lax.fori_loop(0, 640 // 16, frow, 0)

        def orow(i, _):
            ones1[pl.ds(i * 16, 16)] = one16
            return 0
        lax.fori_loop(0, grp // 16, orow, 0)

        ich = (ng // 13) * grp

        def iload(i, _):
            pltpu.sync_copy(dst_h.at[pl.ds(gb * grp + i * ich, ich)],
                            idx_all.at[pl.ds(i * ich, ich)])
            return 0
        lax.fori_loop(0, 13, iload, 0)

        @pl.when(has_x)
        def _():
            pltpu.sync_copy(dst_h.at[pl.ds(xgrp * grp, grp)],
                            idx_all.at[pl.ds(ng * grp, grp)])

        def route(i, _):
            for k in range(grp // 16):
                iv = idx_all[pl.ds(i * grp + k * 16, 16)]
                t = iv - lo_c
                m = (t >= 0) & (t < half)
                idx2_all[i, pl.ds(k * 16, 16)] = jnp.where(
                    m, t, half + (iv & 31))
            return 0
        lax.fori_loop(0, ng + 1, route, 0)

        zbase = sid * zrt

        def zero_acc():
            pltpu.sync_copy(zbuf, acc.at[pl.ds(zbase, zr)])
            pltpu.sync_copy(zbuf, acc.at[pl.ds(zbase + zr, zr)])

        def run_layer(he_h, s_o, do_deg):
            def body(g, _):
                pltpu.sync_copy(he_h.at[pl.ds((gb + g) * grp, grp)],
                                rows_a)
                pltpu.sync_copy(rows_a, acc.at[idx2_all.at[g]], add=True)
                if do_deg:
                    pltpu.sync_copy(ones1, dacc.at[idx2_all.at[g]],
                                    add=True)
                return 0

            lax.fori_loop(0, ng, body, 0)

            @pl.when(has_x)
            def _():
                pltpu.sync_copy(he_h.at[pl.ds(xgrp * grp, grp)], rows_a)
                pltpu.sync_copy(rows_a, acc.at[idx2_all.at[ng]], add=True)
                if do_deg:
                    pltpu.sync_copy(ones1, dacc.at[idx2_all.at[ng]],
                                    add=True)

            plsc.subcore_barrier()
            pltpu.sync_copy(acc.at[pl.ds(sid * wrt, wrt)],
                            s_o.at[pl.ds(lo_c + sid * wrt, wrt)])

        zero_acc()

        @pl.when(sid < half // 640)
        def _():
            pltpu.sync_copy(dz, dacc.at[pl.ds(sid * 640, 640)])

        @pl.when(sid == 8)
        def _():
            pltpu.sync_copy(dz.at[pl.ds(0, 128)],
                            dacc.at[pl.ds(half, 128)])

        plsc.subcore_barrier()
        run_layer(he1_h, s1_o, True)
        plsc.subcore_barrier()
        zero_acc()
        plsc.subcore_barrier()
        run_layer(he2_h, s2_o, False)

        @pl.when(sid < half // 640)
        def _():
            pltpu.sync_copy(dacc.at[pl.ds(sid * 640, 640)],
                            deg_o.at[pl.ds(lo_c + sid * 640, 640)])

    return seg(he1, he2, dst)


def _node_body(x, s1, s2, deg,
               ew2t1, eb2_1, nwxt1, nwet1, nb1,
               ew2t2, eb2_2, nwxt2, nwet2, nb2,
               muwt, mub, lvwt, lvb, mu, lv):
    def dot(a, b):
        return jnp.dot(a, b[...], preferred_element_type=jnp.float32,
                       precision=lax.Precision.HIGHEST)

    d = deg[...]
    u1 = dot(s1[...], ew2t1) + d * eb2_1[...]
    agg1 = d * dot(x[...], nwxt1) + dot(u1, nwet1) + d * nb1[...]
    h1 = _leaky(agg1)
    u2 = dot(s2[...], ew2t2) + d * eb2_2[...]
    agg2 = d * dot(h1, nwxt2) + dot(u2, nwet2) + d * nb2[...]
    h2 = _leaky(agg2)
    mu[...] = dot(h2, muwt) + mub[...]
    lv[...] = dot(h2, lvwt) + lvb[...]


def _node_stage(x, s1, s2, deg, wts, bn=1000):
    N, D = x.shape
    H = s1.shape[1]
    L = wts["muwt"].shape[1]
    grid = N // bn

    def full(shape):
        return pl.BlockSpec(shape, lambda i: (0, 0))

    return pl.pallas_call(
        _node_body,
        grid=(grid,),
        in_specs=[
            pl.BlockSpec((bn, D), lambda i: (i, 0)),
            pl.BlockSpec((bn, H), lambda i: (i, 0)),
            pl.BlockSpec((bn, H), lambda i: (i, 0)),
            pl.BlockSpec((bn, 1), lambda i: (i, 0)),
            full((H, H)), full((1, H)), full((D, H)), full((H, H)),
            full((1, H)),
            full((H, H)), full((1, H)), full((H, H)), full((H, H)),
            full((1, H)),
            full((H, L)), full((1, L)), full((H, L)), full((1, L)),
        ],
        out_specs=[
            pl.BlockSpec((bn, L), lambda i: (i, 0)),
            pl.BlockSpec((bn, L), lambda i: (i, 0)),
        ],
        out_shape=[
            jax.ShapeDtypeStruct((N, L), jnp.float32),
            jax.ShapeDtypeStruct((N, L), jnp.float32),
        ],
    )(x, s1, s2, deg,
      wts["ew2t1"], wts["eb2_1"], wts["nwxt1"], wts["nwet1"], wts["nb1"],
      wts["ew2t2"], wts["eb2_2"], wts["nwxt2"], wts["nwet2"], wts["nb2"],
      wts["muwt"], wts["mub"], wts["lvwt"], wts["lvb"])


def kernel(x, edge_index, edge_attr,
           c1_ew1, c1_eb1, c1_ew2, c1_eb2, c1_nw, c1_nb,
           c2_ew1, c2_eb1, c2_ew2, c2_eb2, c2_nw, c2_nb,
           mu_w, mu_b, lv_w, lv_b):
    N, D = x.shape
    H = c1_ew1.shape[0]
    dst = edge_index[1].astype(jnp.int32)

    he1, he2 = _edge_mlp(edge_attr,
                         c1_ew1.T, c1_eb1[None, :],
                         c2_ew1.T, c2_eb1[None, :])

    npad = ((N + 2047) // 2048) * 2048
    s1f, s2f, degf = _segsum_sc(he1, he2, dst, npad)
    s1 = s1f[:N]
    s2 = s2f[:N]
    deg = degf[:N][:, None]

    wts = dict(
        ew2t1=c1_ew2.T, eb2_1=c1_eb2[None, :],
        nwxt1=c1_nw[:, :D].T, nwet1=c1_nw[:, D:].T, nb1=c1_nb[None, :],
        ew2t2=c2_ew2.T, eb2_2=c2_eb2[None, :],
        nwxt2=c2_nw[:, :H].T, nwet2=c2_nw[:, H:].T, nb2=c2_nb[None, :],
        muwt=mu_w.T, mub=mu_b[None, :],
        lvwt=lv_w.T, lvb=lv_b[None, :],
    )
    mu, lv = _node_stage(x, s1, s2, deg, wts)
    return mu, lv


# speedup vs baseline: 3.5003x; 1.0550x over previous
import functools

import jax
import jax.numpy as jnp
from jax import lax
from jax.experimental import pallas as pl
from jax.experimental.pallas import tpu as pltpu
from jax.experimental.pallas import tpu_sc as plsc


def _leaky(v):
    return jnp.where(v >= 0, v, 0.15 * v)


def _he_body1(ea, w1, b1, he1):
    e = ea[...]
    h1 = jnp.dot(e, w1[...], preferred_element_type=jnp.float32,
                 precision=lax.Precision.HIGHEST) + b1[...]
    he1[...] = _leaky(h1)


def _edge_mlp1(ea, w1t, b1, be=3200):
    E, DE = ea.shape
    H = w1t.shape[1]
    grid = E // be
    return pl.pallas_call(
        _he_body1,
        grid=(grid,),
        in_specs=[
            pl.BlockSpec((be, DE), lambda i: (i, 0)),
            pl.BlockSpec((DE, H), lambda i: (0, 0)),
            pl.BlockSpec((1, H), lambda i: (0, 0)),
        ],
        out_specs=pl.BlockSpec((be, H), lambda i: (i, 0)),
        out_shape=jax.ShapeDtypeStruct((E, H), jnp.float32),
    )(ea, w1t, b1)


_CH = 80
_DW = 16


def _segsum_one(he, dst, npad, with_deg):
    E, H = he.shape
    ns = 16
    half = npad // 2
    arows = half + 32
    zrt = arows // ns
    wrt = half // ns
    grp = 128
    tg = E // grp
    ng = tg // ns
    xg = tg - ng * ns
    zr = zrt // 2

    mesh = plsc.VectorSubcoreMesh(core_axis_name="c", subcore_axis_name="s")
    out_type = [jax.ShapeDtypeStruct((npad, H), jnp.float32)]
    scratch = [
        pltpu.VMEM(((ng + 1) * grp,), jnp.int32),
        pltpu.VMEM((ng + 1, grp), jnp.int32),
        pltpu.VMEM((grp, H), jnp.float32),
        pltpu.VMEM((zr, H), jnp.float32),
        pltpu.VMEM_SHARED((arows, H), jnp.float32),
    ]
    if with_deg:
        out_type.append(jax.ShapeDtypeStruct((npad,), jnp.float32))
        scratch += [
            pltpu.VMEM((grp,), jnp.float32),
            pltpu.VMEM((640,), jnp.float32),
            pltpu.VMEM_SHARED((half + 128,), jnp.float32),
        ]

    @functools.partial(pl.kernel, out_type=out_type, mesh=mesh,
                       scratch_types=scratch)
    def seg(he_h, dst_h, *refs):
        if with_deg:
            (s_o, deg_o, idx_all, idx2_all, rows_a, zbuf, acc,
             ones1, dz, dacc) = refs
        else:
            (s_o, idx_all, idx2_all, rows_a, zbuf, acc) = refs
        cid = lax.axis_index("c")
        sid = lax.axis_index("s")
        zero16 = jnp.zeros((16,), jnp.float32)
        one16 = jnp.ones((16,), jnp.float32)
        lo_c = cid * half
        gb = sid * ng
        xgrp = ns * ng + sid
        has_x = sid < xg

        def zrow(i, _):
            for k in range(H // 16):
                zbuf[i, pl.ds(k * 16, 16)] = zero16
            return 0
        lax.fori_loop(0, zr, zrow, 0)

        if with_deg:
            def frow(i, _):
                dz[pl.ds(i * 16, 16)] = zero16
                return 0
            lax.fori_loop(0, 640 // 16, frow, 0)

            def orow(i, _):
                ones1[pl.ds(i * 16, 16)] = one16
                return 0
            lax.fori_loop(0, grp // 16, orow, 0)

        ich = (ng // 13) * grp

        def iload(i, _):
            pltpu.sync_copy(dst_h.at[pl.ds(gb * grp + i * ich, ich)],
                            idx_all.at[pl.ds(i * ich, ich)])
            return 0
        lax.fori_loop(0, 13, iload, 0)

        @pl.when(has_x)
        def _():
            pltpu.sync_copy(dst_h.at[pl.ds(xgrp * grp, grp)],
                            idx_all.at[pl.ds(ng * grp, grp)])

        def route(i, _):
            for k in range(grp // 16):
                iv = idx_all[pl.ds(i * grp + k * 16, 16)]
                t = iv - lo_c
                m = (t >= 0) & (t < half)
                idx2_all[i, pl.ds(k * 16, 16)] = jnp.where(
                    m, t, half + (iv & 31))
            return 0
        lax.fori_loop(0, ng + 1, route, 0)

        zbase = sid * zrt
        pltpu.sync_copy(zbuf, acc.at[pl.ds(zbase, zr)])
        pltpu.sync_copy(zbuf, acc.at[pl.ds(zbase + zr, zr)])

        if with_deg:
            @pl.when(sid < half // 640)
            def _():
                pltpu.sync_copy(dz, dacc.at[pl.ds(sid * 640, 640)])

            @pl.when(sid == 8)
            def _():
                pltpu.sync_copy(dz.at[pl.ds(0, 128)],
                                dacc.at[pl.ds(half, 128)])

        plsc.subcore_barrier()

        def body(g, _):
            pltpu.sync_copy(he_h.at[pl.ds((gb + g) * grp, grp)], rows_a)
            pltpu.sync_copy(rows_a, acc.at[idx2_all.at[g]], add=True)
            if with_deg:
                pltpu.sync_copy(ones1, dacc.at[idx2_all.at[g]], add=True)
            return 0

        lax.fori_loop(0, ng, body, 0)

        @pl.when(has_x)
        def _():
            pltpu.sync_copy(he_h.at[pl.ds(xgrp * grp, grp)], rows_a)
            pltpu.sync_copy(rows_a, acc.at[idx2_all.at[ng]], add=True)
            if with_deg:
                pltpu.sync_copy(ones1, dacc.at[idx2_all.at[ng]], add=True)

        plsc.subcore_barrier()

        pltpu.sync_copy(acc.at[pl.ds(sid * wrt, wrt)],
                        s_o.at[pl.ds(lo_c + sid * wrt, wrt)])
        if with_deg:
            @pl.when(sid < half // 640)
            def _():
                pltpu.sync_copy(dacc.at[pl.ds(sid * 640, 640)],
                                deg_o.at[pl.ds(lo_c + sid * 640, 640)])

    return seg(he, dst)


def _node_body(x, s1, s2, deg,
               ew2t1, eb2_1, nwxt1, nwet1, nb1,
               ew2t2, eb2_2, nwxt2, nwet2, nb2,
               muwt, mub, lvwt, lvb, mu, lv):
    def dot(a, b):
        return jnp.dot(a, b[...], preferred_element_type=jnp.float32,
                       precision=lax.Precision.HIGHEST)

    d = deg[...]
    u1 = dot(s1[...], ew2t1) + d * eb2_1[...]
    agg1 = d * dot(x[...], nwxt1) + dot(u1, nwet1) + d * nb1[...]
    h1 = _leaky(agg1)
    u2 = dot(s2[...], ew2t2) + d * eb2_2[...]
    agg2 = d * dot(h1, nwxt2) + dot(u2, nwet2) + d * nb2[...]
    h2 = _leaky(agg2)
    mu[...] = dot(h2, muwt) + mub[...]
    lv[...] = dot(h2, lvwt) + lvb[...]


def _node_stage(x, s1, s2, deg, wts, bn=1000):
    N, D = x.shape
    H = s1.shape[1]
    L = wts["muwt"].shape[1]
    grid = N // bn

    def full(shape):
        return pl.BlockSpec(shape, lambda i: (0, 0))

    return pl.pallas_call(
        _node_body,
        grid=(grid,),
        in_specs=[
            pl.BlockSpec((bn, D), lambda i: (i, 0)),
            pl.BlockSpec((bn, H), lambda i: (i, 0)),
            pl.BlockSpec((bn, H), lambda i: (i, 0)),
            pl.BlockSpec((bn, 1), lambda i: (i, 0)),
            full((H, H)), full((1, H)), full((D, H)), full((H, H)),
            full((1, H)),
            full((H, H)), full((1, H)), full((H, H)), full((H, H)),
            full((1, H)),
            full((H, L)), full((1, L)), full((H, L)), full((1, L)),
        ],
        out_specs=[
            pl.BlockSpec((bn, L), lambda i: (i, 0)),
            pl.BlockSpec((bn, L), lambda i: (i, 0)),
        ],
        out_shape=[
            jax.ShapeDtypeStruct((N, L), jnp.float32),
            jax.ShapeDtypeStruct((N, L), jnp.float32),
        ],
    )(x, s1, s2, deg,
      wts["ew2t1"], wts["eb2_1"], wts["nwxt1"], wts["nwet1"], wts["nb1"],
      wts["ew2t2"], wts["eb2_2"], wts["nwxt2"], wts["nwet2"], wts["nb2"],
      wts["muwt"], wts["mub"], wts["lvwt"], wts["lvb"])


def kernel(x, edge_index, edge_attr,
           c1_ew1, c1_eb1, c1_ew2, c1_eb2, c1_nw, c1_nb,
           c2_ew1, c2_eb1, c2_ew2, c2_eb2, c2_nw, c2_nb,
           mu_w, mu_b, lv_w, lv_b):
    N, D = x.shape
    H = c1_ew1.shape[0]
    dst = edge_index[1].astype(jnp.int32)

    npad = ((N + 2047) // 2048) * 2048
    he1 = _edge_mlp1(edge_attr, c1_ew1.T, c1_eb1[None, :])
    s1f, degf = _segsum_one(he1, dst, npad, True)
    he2 = _edge_mlp1(edge_attr, c2_ew1.T, c2_eb1[None, :])
    (s2f,) = _segsum_one(he2, dst, npad, False)
    s1 = s1f[:N]
    s2 = s2f[:N]
    deg = degf[:N][:, None]

    wts = dict(
        ew2t1=c1_ew2.T, eb2_1=c1_eb2[None, :],
        nwxt1=c1_nw[:, :D].T, nwet1=c1_nw[:, D:].T, nb1=c1_nb[None, :],
        ew2t2=c2_ew2.T, eb2_2=c2_eb2[None, :],
        nwxt2=c2_nw[:, :H].T, nwet2=c2_nw[:, H:].T, nb2=c2_nb[None, :],
        muwt=mu_w.T, mub=mu_b[None, :],
        lvwt=lv_w.T, lvb=lv_b[None, :],
    )
    mu, lv = _node_stage(x, s1, s2, deg, wts)
    return mu, lv
